# Initial kernel scaffold; baseline (speedup 1.0000x reference)
import jax
import jax.numpy as jnp
from jax.experimental import pallas as pl


def kernel(self_embs, neigh_embs, W_self, b_self, W_neigh, b_neigh):
    raise NotImplementedError("write your pallas kernel here")


# speedup vs baseline: 1.2713x; 1.2713x over previous
import jax
import jax.numpy as jnp
from jax.experimental import pallas as pl
from jax.experimental.pallas import tpu as pltpu

N = 10000
K = 32
D_IN = 128
D_OUT = 128
BN = 400


def _body(self_ref, neigh_ref, wts_ref, wtn_ref, b_ref, out_ref):
    neigh_mean = jnp.sum(neigh_ref[...], axis=1) * (1.0 / K)
    t = jnp.dot(self_ref[...], wts_ref[...], preferred_element_type=jnp.float32)
    t = t + jnp.dot(neigh_mean, wtn_ref[...], preferred_element_type=jnp.float32)
    t = t + b_ref[...]
    c = jnp.maximum(t, 0.0)
    norm2 = jnp.sum(c * c, axis=1, keepdims=True)
    out_ref[...] = c * jax.lax.rsqrt(jnp.maximum(norm2, 1e-24))


def kernel(self_embs, neigh_embs, W_self, b_self, W_neigh, b_neigh):
    wts = W_self.T
    wtn = W_neigh.T
    b = (b_self + b_neigh).reshape(1, D_OUT)
    grid = (N // BN,)
    return pl.pallas_call(
        _body,
        grid=grid,
        in_specs=[
            pl.BlockSpec((BN, D_IN), lambda i: (i, 0)),
            pl.BlockSpec((BN, K, D_IN), lambda i: (i, 0, 0)),
            pl.BlockSpec((D_IN, D_OUT), lambda i: (0, 0)),
            pl.BlockSpec((D_IN, D_OUT), lambda i: (0, 0)),
            pl.BlockSpec((1, D_OUT), lambda i: (0, 0)),
        ],
        out_specs=pl.BlockSpec((BN, D_OUT), lambda i: (i, 0)),
        out_shape=jax.ShapeDtypeStruct((N, D_OUT), jnp.float32),
        compiler_params=pltpu.CompilerParams(
            dimension_semantics=("arbitrary",),
        ),
    )(self_embs, neigh_embs, wts, wtn, b)
